# Initial kernel scaffold; baseline (speedup 1.0000x reference)
import jax
import jax.numpy as jnp
from jax.experimental import pallas as pl


def kernel(output, nh_indices):
    raise NotImplementedError("write your pallas kernel here")


# speedup vs baseline: 5.6114x; 5.6114x over previous
import functools

import jax
import jax.numpy as jnp
from jax import lax
from jax.experimental import pallas as pl
from jax.experimental.pallas import tpu as pltpu
from jax.experimental.pallas import tpu_sc as plsc

_C = 32


@functools.lru_cache(maxsize=None)
def _make_sc_kernel(rows_total: int, d: int, nchunk: int, cnt_max: int,
                    base_cnt: int, rem: int, nk: int):
    mesh = plsc.VectorSubcoreMesh(core_axis_name="c", subcore_axis_name="s",
                                  num_cores=2, num_subcores=16)
    nc = mesh.num_cores
    nw = nc * mesh.num_subcores
    nv = d // 16

    @functools.partial(
        pl.kernel,
        out_type=jax.ShapeDtypeStruct((nw, 16), jnp.float32),
        mesh=mesh,
        compiler_params=pltpu.CompilerParams(use_tc_tiling_on_sc=False),
        scratch_types=[
            pltpu.VMEM((cnt_max, nk, _C), jnp.int32),
            pltpu.VMEM((_C, d), jnp.float32),
            pltpu.VMEM((nk, _C, d), jnp.float32),
            pltpu.VMEM((16,), jnp.float32),
            pltpu.SemaphoreType.DMA,
        ],
    )
    def launch(table, idx_tiles, out, idx_v, cbuf, nbuf, accv, sem):
        wid = lax.axis_index("s") * nc + lax.axis_index("c")
        start = wid * base_cnt + jnp.minimum(wid, rem)
        cnt = base_cnt + (wid < rem).astype(jnp.int32)

        pltpu.sync_copy(idx_tiles.at[wid], idx_v)

        def chunk_body(j, accs):
            base = (start + j) * _C
            hc = pltpu.async_copy(table.at[pl.ds(base, _C)], cbuf, sem)
            hs = [pltpu.async_copy(table.at[idx_v.at[j, k]], nbuf.at[k], sem)
                  for k in range(nk)]
            hc.wait()
            for h in hs:
                h.wait()

            def row_body(r, a):
                ctr = [cbuf[r, pl.ds(16 * v, 16)] for v in range(nv)]
                a = list(a)
                for k in range(nk):
                    for v in range(nv):
                        a[v] = a[v] + jnp.abs(
                            nbuf[k, r, pl.ds(16 * v, 16)] - ctr[v])
                return tuple(a)

            return lax.fori_loop(0, _C, row_body, accs)

        zeros = jnp.zeros((16,), jnp.float32)
        accs = lax.fori_loop(0, cnt, chunk_body, (zeros,) * nv)
        tot = accs[0]
        for v in range(1, nv):
            tot = tot + accs[v]
        accv[...] = tot
        pltpu.sync_copy(accv, out.at[wid])

    return launch, nw


def kernel(output, nh_indices):
    b, n, d = output.shape
    k_all = nh_indices.shape[1]
    nk = k_all - 1
    rows_total = b * n
    assert rows_total % _C == 0
    nchunk = rows_total // _C

    table = output.reshape(rows_total, d)
    nh = nh_indices[:, 1:].astype(jnp.int32)
    idx = jnp.arange(b, dtype=jnp.int32)[:, None, None] * n + nh[None]
    idx = idx.reshape(nchunk, _C, nk).transpose(0, 2, 1)

    base_cnt, rem = nchunk // 32, nchunk % 32
    cnt_max = base_cnt + (1 if rem else 0)
    launch, nw = _make_sc_kernel(rows_total, d, nchunk, cnt_max, base_cnt, rem, nk)

    pad = nw * cnt_max - nchunk
    idx_pad = jnp.concatenate(
        [idx, jnp.zeros((pad, nk, _C), jnp.int32)], axis=0) if pad else idx
    starts = jnp.arange(nw, dtype=jnp.int32) * base_cnt + jnp.minimum(
        jnp.arange(nw, dtype=jnp.int32), rem)
    gidx = starts[:, None] + jnp.arange(cnt_max, dtype=jnp.int32)[None, :]
    idx_tiles = jnp.take(idx_pad, gidx, axis=0)

    partials = launch(table, idx_tiles)
    return jnp.sqrt(jnp.sum(partials) / (rows_total * nk * d))


# speedup vs baseline: 8.9886x; 1.6018x over previous
import functools

import jax
import jax.numpy as jnp
from jax import lax
from jax.experimental import pallas as pl
from jax.experimental.pallas import tpu as pltpu
from jax.experimental.pallas import tpu_sc as plsc

_C = 32


@functools.lru_cache(maxsize=None)
def _make_sc_kernel(rows_total: int, d: int, nchunk: int, cnt_max: int,
                    base_cnt: int, rem: int, nk: int):
    mesh = plsc.VectorSubcoreMesh(core_axis_name="c", subcore_axis_name="s",
                                  num_cores=2, num_subcores=16)
    nc = mesh.num_cores
    nw = nc * mesh.num_subcores
    nv = d // 16

    @functools.partial(
        pl.kernel,
        out_type=jax.ShapeDtypeStruct((nw, 16), jnp.float32),
        mesh=mesh,
        compiler_params=pltpu.CompilerParams(use_tc_tiling_on_sc=False),
        scratch_types=[
            pltpu.VMEM((cnt_max, nk, _C), jnp.int32),
            pltpu.VMEM((2, _C, d), jnp.float32),
            pltpu.VMEM((2, nk, _C, d), jnp.float32),
            pltpu.VMEM((16,), jnp.float32),
            pltpu.SemaphoreType.DMA,
            pltpu.SemaphoreType.DMA,
        ],
    )
    def launch(table, idx_tiles, out, idx_v, cbuf, nbuf, accv, sem0, sem1):
        wid = lax.axis_index("s") * nc + lax.axis_index("c")
        start = wid * base_cnt + jnp.minimum(wid, rem)
        cnt = base_cnt + (wid < rem).astype(jnp.int32)
        sems = (sem0, sem1)

        pltpu.sync_copy(idx_tiles.at[wid], idx_v)
        accv[...] = jnp.zeros((16,), jnp.float32)

        def issue(j, p):
            base = (start + j) * _C
            pltpu.async_copy(table.at[pl.ds(base, _C)], cbuf.at[p], sems[p])
            for k in range(nk):
                pltpu.async_copy(table.at[idx_v.at[j, k]], nbuf.at[p, k],
                                 sems[p])

        def wait_chunk(j, p):
            base = (start + j) * _C
            pltpu.make_async_copy(
                table.at[pl.ds(base, _C)], cbuf.at[p], sems[p]).wait()
            for k in range(nk):
                pltpu.make_async_copy(
                    table.at[idx_v.at[j, k]], nbuf.at[p, k], sems[p]).wait()

        def compute(p):
            def row_body(r, a):
                ctr = [cbuf[p, r, pl.ds(16 * v, 16)] for v in range(nv)]
                a = list(a)
                for k in range(nk):
                    for v in range(nv):
                        a[v] = a[v] + jnp.abs(
                            nbuf[p, k, r, pl.ds(16 * v, 16)] - ctr[v])
                return tuple(a)

            zeros = jnp.zeros((16,), jnp.float32)
            accs = lax.fori_loop(0, _C, row_body, (zeros,) * nv)
            tot = accs[0]
            for v in range(1, nv):
                tot = tot + accs[v]
            accv[...] = accv[...] + tot

        issue(0, 0)

        def body2(jj, _):
            j0 = jj * 2

            @pl.when(j0 + 1 < cnt)
            def _():
                issue(j0 + 1, 1)

            wait_chunk(j0, 0)
            compute(0)

            @pl.when(j0 + 2 < cnt)
            def _():
                issue(j0 + 2, 0)

            @pl.when(j0 + 1 < cnt)
            def _():
                wait_chunk(j0 + 1, 1)
                compute(1)

            return 0

        lax.fori_loop(0, (cnt + 1) // 2, body2, 0)
        pltpu.sync_copy(accv, out.at[wid])

    return launch, nw


def kernel(output, nh_indices):
    b, n, d = output.shape
    k_all = nh_indices.shape[1]
    nk = k_all - 1
    rows_total = b * n
    assert rows_total % _C == 0
    nchunk = rows_total // _C

    table = output.reshape(rows_total, d)
    nh = nh_indices[:, 1:].astype(jnp.int32)
    idx = jnp.arange(b, dtype=jnp.int32)[:, None, None] * n + nh[None]
    idx = idx.reshape(nchunk, _C, nk).transpose(0, 2, 1)

    base_cnt, rem = nchunk // 32, nchunk % 32
    cnt_max = base_cnt + (1 if rem else 0)
    launch, nw = _make_sc_kernel(rows_total, d, nchunk, cnt_max, base_cnt, rem, nk)

    pad = nw * cnt_max - nchunk
    idx_pad = jnp.concatenate(
        [idx, jnp.zeros((pad, nk, _C), jnp.int32)], axis=0) if pad else idx
    starts = jnp.arange(nw, dtype=jnp.int32) * base_cnt + jnp.minimum(
        jnp.arange(nw, dtype=jnp.int32), rem)
    gidx = starts[:, None] + jnp.arange(cnt_max, dtype=jnp.int32)[None, :]
    idx_tiles = jnp.take(idx_pad, gidx, axis=0)

    partials = launch(table, idx_tiles)
    return jnp.sqrt(jnp.sum(partials) / (rows_total * nk * d))
